# baseline (speedup 1.0000x reference)
import functools

import jax
import jax.numpy as jnp
from jax import lax
from jax.experimental import pallas as pl
from jax.experimental.pallas import tpu as pltpu
from jax.experimental.pallas import tpu_sc as plsc

B, L, D, CLS = 4096, 200, 64, 2
NC, NS = 2, 16
NW = NC * NS
BPW = B // NW
CHUNK = 4
NCH = BPW // CHUNK
ROWS = CHUNK * L
NSUB = 8
SUB = ROWS // NSUB
INV_L = 1.0 / L


def _pool_body(x_hbm, table_hbm, out_hbm,
               idx0, idx1, rows0, rows1, stage, sem0, sem1):
    wid = lax.axis_index("s") * NC + lax.axis_index("c")
    base_chunk = wid * NCH
    idx = (idx0, idx1)
    rows = (rows0, rows1)
    sems = (sem0, sem1)

    def issue(j, p):
        pltpu.sync_copy(x_hbm.at[base_chunk + j], idx[p])
        for k in range(NSUB):
            pltpu.async_copy(table_hbm.at[idx[p].at[k]],
                             rows[p].at[pl.ds(k * SUB, SUB)], sems[p])

    def wait(p):
        pltpu.make_async_copy(table_hbm.at[pl.ds(0, ROWS)], rows[p],
                              sems[p]).wait()

    def compute(j, p):
        r = rows[p]
        for e in range(CHUNK):
            base = e * L
            zero = jnp.zeros((16,), jnp.float32)

            def body(l, accs, base=base):
                a0, a1, a2, a3 = accs
                rr = base + 2 * l
                a0 = a0 + r[rr, pl.ds(0, 16)] + r[rr + 1, pl.ds(0, 16)]
                a1 = a1 + r[rr, pl.ds(16, 16)] + r[rr + 1, pl.ds(16, 16)]
                a2 = a2 + r[rr, pl.ds(32, 16)] + r[rr + 1, pl.ds(32, 16)]
                a3 = a3 + r[rr, pl.ds(48, 16)] + r[rr + 1, pl.ds(48, 16)]
                return a0, a1, a2, a3

            a0, a1, a2, a3 = lax.fori_loop(0, L // 2, body, (zero,) * 4)
            stage[e, pl.ds(0, 16)] = a0 * INV_L
            stage[e, pl.ds(16, 16)] = a1 * INV_L
            stage[e, pl.ds(32, 16)] = a2 * INV_L
            stage[e, pl.ds(48, 16)] = a3 * INV_L
        row0 = wid * BPW + j * CHUNK
        pltpu.sync_copy(stage, out_hbm.at[pl.ds(row0, CHUNK)])

    issue(0, 0)

    def outer(i, carry):
        j = 2 * i
        wait(0)
        issue(j + 1, 1)
        compute(j, 0)
        wait(1)

        @pl.when(j + 2 < NCH)
        def _():
            issue(j + 2, 0)

        compute(j + 1, 1)
        return carry

    lax.fori_loop(0, NCH // 2, outer, 0)


_pool = pl.kernel(
    _pool_body,
    mesh=plsc.VectorSubcoreMesh(core_axis_name="c", subcore_axis_name="s"),
    out_type=jax.ShapeDtypeStruct((B, D), jnp.float32),
    compiler_params=pltpu.CompilerParams(use_tc_tiling_on_sc=False),
    scratch_types=[
        pltpu.VMEM((NSUB, SUB), jnp.int32),
        pltpu.VMEM((NSUB, SUB), jnp.int32),
        pltpu.VMEM((ROWS, D), jnp.float32),
        pltpu.VMEM((ROWS, D), jnp.float32),
        pltpu.VMEM((CHUNK, D), jnp.float32),
        pltpu.SemaphoreType.DMA,
        pltpu.SemaphoreType.DMA,
    ],
)


def _head_body(p_ref, w_ref, b_ref, o_ref):
    o_ref[...] = jnp.dot(p_ref[...], w_ref[...],
                         preferred_element_type=jnp.float32) + b_ref[...]


def _head(pooled, W, b):
    return pl.pallas_call(
        _head_body,
        out_shape=jax.ShapeDtypeStruct((B, CLS), jnp.float32),
    )(pooled, W.T, b.reshape(1, CLS))


def kernel(x, table, W, b):
    x32 = x.astype(jnp.int32).reshape(NW * NCH, NSUB, SUB)
    pooled = _pool(x32, table)
    return _head(pooled, W, b)
